# baseline (speedup 1.0000x reference)
import functools

import jax
import jax.numpy as jnp
from jax import lax
from jax.experimental import pallas as pl
from jax.experimental.pallas import tpu as pltpu
from jax.experimental.pallas import tpu_sc as plsc

N = 50000
H = 64
E = 800000
B = 200000
NLAYERS = 2
EPS = 1e-5

HALF = 25000
TRASH = HALF
HPAD = 25088
NPAD = 2 * HPAD
PADW = HPAD - HALF

NC = 2
NS = 16
G = 128
EG = E // G
GSUB = 392
EGP = GSUB * NS
SUPG = 28
NSUP = GSUB // SUPG
ZS = HPAD // NS
CW = 16

BG = 1568
BPAD = BG * G
FG_PW = BG // (NC * NS)

_mesh = plsc.VectorSubcoreMesh(core_axis_name="c", subcore_axis_name="s")
_SC_PARAMS = pltpu.CompilerParams(use_tc_tiling_on_sc=False)
_SC_PARAMS_NL = pltpu.CompilerParams(use_tc_tiling_on_sc=False,
                                     needs_layout_passes=False)


def _remap_src_row(row_ref):
    for i in range(G // 16):
        v = row_ref[pl.ds(i * 16, 16)]
        row_ref[pl.ds(i * 16, 16)] = jnp.where(v >= HALF, v + PADW, v)


def _remap_dst_row(row_ref, c):
    base = c * HALF
    for i in range(G // 16):
        v = row_ref[pl.ds(i * 16, 16)] - base
        ok = (v >= 0) & (v < HALF)
        row_ref[pl.ds(i * 16, 16)] = jnp.where(ok, v, TRASH)


def _remap_pair_row(src_ref, dst_ref, c):
    base = c * HALF
    for i in range(G // 16):
        sl = pl.ds(i * 16, 16)
        d = dst_ref[sl] - base
        ok = (d >= 0) & (d < HALF)
        dst_ref[sl] = jnp.where(ok, d, TRASH)
        v = src_ref[sl]
        v = jnp.where(v >= HALF, v + PADW, v)
        src_ref[sl] = jnp.where(ok, v, TRASH)
    return None


def _zero_and_barrier(zeros_hbm, acc_sh, s):
    pltpu.sync_copy(zeros_hbm, acc_sh.at[pl.ds(s * ZS, ZS)])
    plsc.subcore_barrier()


def _dump_half(acc_sh, out_hbm, c, s):
    pltpu.sync_copy(acc_sh.at[pl.ds(s * ZS, ZS)],
                    out_hbm.at[c, pl.ds(s * ZS, ZS)])


def _agg_body(src_hbm, dst_hbm, table_hbm, zeros_hbm, out_hbm,
              acc_sh, sidx, ldst, rows,
              isem0, isem1, gsem0, gsem1, ssem0, ssem1):
    c = lax.axis_index("c")
    s = lax.axis_index("s")
    _zero_and_barrier(zeros_hbm, acc_sh, s)
    isems = (isem0, isem1)
    gsems = (gsem0, gsem1)
    ssems = (ssem0, ssem1)
    g0row = s * GSUB

    def idx_start(p, pp):
        base = g0row + p * SUPG
        pltpu.async_copy(src_hbm.at[pl.ds(base, SUPG)], sidx.at[pp],
                         isems[pp])
        pltpu.async_copy(dst_hbm.at[pl.ds(base, SUPG)], ldst.at[pp],
                         isems[pp])

    def idx_wait(p, pp):
        base = g0row + p * SUPG
        pltpu.make_async_copy(src_hbm.at[pl.ds(base, SUPG)], sidx.at[pp],
                              isems[pp]).wait()
        pltpu.make_async_copy(dst_hbm.at[pl.ds(base, SUPG)], ldst.at[pp],
                              isems[pp]).wait()

    def gather_start(rr, pp, r):
        pltpu.async_copy(table_hbm.at[sidx.at[pp, r]], rows.at[rr],
                         gsems[rr])

    def gather_wait(rr, pp, r):
        pltpu.make_async_copy(table_hbm.at[sidx.at[pp, r]], rows.at[rr],
                              gsems[rr]).wait()

    def scatter_start(rr, pp, r):
        pltpu.async_copy(rows.at[rr], acc_sh.at[ldst.at[pp, r]], ssems[rr],
                         add=True)

    def scatter_wait(rr, pp, r):
        pltpu.make_async_copy(rows.at[rr], acc_sh.at[ldst.at[pp, r]],
                              ssems[rr]).wait()

    def do_chunk(p, pp):
        idx_wait(p, pp)

        def rbody(r, carry):
            _remap_pair_row(sidx.at[pp, r], ldst.at[pp, r], c)
            return carry

        lax.fori_loop(0, SUPG, rbody, 0)

        def gbody(r0, carry):
            for rr in (0, 1):
                r = r0 * 2 + rr

                @pl.when(r0 >= 1)
                def _():
                    scatter_wait(rr, pp, r - 2)

                gather_start(rr, pp, r)
                gather_wait(rr, pp, r)
                scatter_start(rr, pp, r)
            return carry

        lax.fori_loop(0, SUPG // 2, gbody, 0)
        scatter_wait(0, pp, SUPG - 2)
        scatter_wait(1, pp, SUPG - 1)

        @pl.when(p < NSUP - 2)
        def _():
            idx_start(p + 2, pp)

    idx_start(0, 0)
    idx_start(1, 1)

    def pbody(p0, carry):
        do_chunk(p0 * 2, 0)
        do_chunk(p0 * 2 + 1, 1)
        return carry

    lax.fori_loop(0, NSUP // 2, pbody, 0)
    plsc.subcore_barrier()
    _dump_half(acc_sh, out_hbm, c, s)


_agg = functools.partial(
    pl.kernel,
    out_type=jax.ShapeDtypeStruct((NC, HPAD, H), jnp.float32),
    mesh=_mesh,
    compiler_params=_SC_PARAMS,
    scratch_types=[
        pltpu.VMEM_SHARED((HPAD, H), jnp.float32),
        pltpu.VMEM((2, SUPG, G), jnp.int32),
        pltpu.VMEM((2, SUPG, G), jnp.int32),
        pltpu.VMEM((2, G, H), jnp.float32),
    ] + [pltpu.SemaphoreType.DMA] * 6,
)(_agg_body)


def _cnt_body(udst_hbm, idst_hbm, outu_hbm, outi_hbm, hist, didx):
    c = lax.axis_index("c")
    s = lax.axis_index("s")
    w = s * NC + c
    ones16 = jnp.ones((16,), jnp.float32)
    gpw = EGP // (NC * NS)

    for dst_hbm, out_hbm in ((udst_hbm, outu_hbm), (idst_hbm, outi_hbm)):
        def zbody(i, carry):
            hist[pl.ds(i * 16, 16)] = jnp.zeros((16,), jnp.float32)
            return carry

        lax.fori_loop(0, NPAD // 16, zbody, 0)

        def cbody(p, carry):
            pltpu.sync_copy(dst_hbm.at[pl.ds(w * gpw + p * SUPG, SUPG)],
                            didx)

            def rbody(r, carry2):
                for i in range(G // 16):
                    v = didx[r, pl.ds(i * 16, 16)]
                    v = jnp.where(v >= HALF, v + PADW, v)
                    plsc.addupdate_scatter(hist, [v], ones16)
                return carry2

            lax.fori_loop(0, SUPG, rbody, 0)
            return carry

        lax.fori_loop(0, gpw // SUPG, cbody, 0)
        pltpu.sync_copy(hist, out_hbm.at[w])


_cnt = functools.partial(
    pl.kernel,
    out_type=(jax.ShapeDtypeStruct((NC * NS, NPAD), jnp.float32),
              jax.ShapeDtypeStruct((NC * NS, NPAD), jnp.float32)),
    mesh=_mesh,
    compiler_params=_SC_PARAMS_NL,
    scratch_types=[
        pltpu.VMEM((NPAD,), jnp.float32),
        pltpu.VMEM((SUPG, G), jnp.int32),
    ],
)(_cnt_body)


def _fgather_body(ui_hbm, ii_hbm, utab_hbm, itab_hbm, uh_hbm, ih_hbm,
                  eidx, rows,
                  gsem0, gsem1, gsem2, gsem3, osem0, osem1, osem2, osem3):
    c = lax.axis_index("c")
    s = lax.axis_index("s")
    w = s * NC + c
    gsems = (gsem0, gsem1, gsem2, gsem3)
    osems = (osem0, osem1, osem2, osem3)
    tabs = (utab_hbm, itab_hbm)
    outs = (uh_hbm, ih_hbm)
    g0row = w * FG_PW

    pltpu.sync_copy(ui_hbm.at[pl.ds(g0row, FG_PW)], eidx.at[0])
    pltpu.sync_copy(ii_hbm.at[pl.ds(g0row, FG_PW)], eidx.at[1])

    def rbody(r, carry):
        _remap_src_row(eidx.at[0, r])
        _remap_src_row(eidx.at[1, r])
        return carry

    lax.fori_loop(0, FG_PW, rbody, 0)

    def gather_start(d, bi, k):
        pltpu.async_copy(tabs[d].at[eidx.at[d, k]], rows.at[bi], gsems[bi])

    def gather_wait(d, bi, k):
        pltpu.make_async_copy(tabs[d].at[eidx.at[d, k]], rows.at[bi],
                              gsems[bi]).wait()

    def write_start(d, bi, k):
        pltpu.async_copy(rows.at[bi], outs[d].at[pl.ds((g0row + k) * G, G)],
                         osems[bi])

    def write_wait(d, bi, k):
        pltpu.make_async_copy(rows.at[bi],
                              outs[d].at[pl.ds((g0row + k) * G, G)],
                              osems[bi]).wait()

    def body(k0, carry):
        for kk in (0, 1):
            k = k0 * 2 + kk
            for d in (0, 1):
                bi = 2 * d + kk
                ob = 2 * d + (1 - kk)

                @pl.when(k0 >= 1)
                def _():
                    write_wait(d, bi, k - 2)

                gather_start(d, bi, k)

                if kk == 0:
                    @pl.when(k0 >= 1)
                    def _():
                        gather_wait(d, ob, k - 1)
                        write_start(d, ob, k - 1)
                else:
                    gather_wait(d, ob, k - 1)
                    write_start(d, ob, k - 1)
        return carry

    lax.fori_loop(0, 24, body, 0)
    for d in (0, 1):
        gather_wait(d, 2 * d + 1, 47)
        write_start(d, 2 * d + 1, 47)
        write_wait(d, 2 * d, 46)
        gather_start(d, 2 * d, 48)
    for d in (0, 1):
        gather_wait(d, 2 * d, 48)
        write_start(d, 2 * d, 48)
    for d in (0, 1):
        write_wait(d, 2 * d + 1, 47)
        write_wait(d, 2 * d, 48)


_fgather = functools.partial(
    pl.kernel,
    out_type=(jax.ShapeDtypeStruct((BPAD, H), jnp.float32),
              jax.ShapeDtypeStruct((BPAD, H), jnp.float32)),
    mesh=_mesh,
    compiler_params=_SC_PARAMS,
    scratch_types=[
        pltpu.VMEM((2, FG_PW, G), jnp.int32),
        pltpu.VMEM((4, G, H), jnp.float32),
    ] + [pltpu.SemaphoreType.DMA] * 8,
)(_fgather_body)


DBLK = 512
MBLK = 1024


def _dense_kernel(acc_ref, cnt_ref, x_ref, a_ref, c_ref, d_ref, o_ref):
    cnt = jnp.sum(cnt_ref[...], axis=0)[:, None]
    mean = acc_ref[...] / jnp.maximum(cnt, 1.0)
    y = (jnp.dot(mean, a_ref[...], preferred_element_type=jnp.float32)
         + jnp.dot(x_ref[...], c_ref[...], preferred_element_type=jnp.float32)
         + d_ref[...])
    o_ref[...] = jnp.maximum(y, 0.0)


def _dense(acc, cnt, x, a, cm, d):
    grid = (NPAD // DBLK,)
    blk = pl.BlockSpec((DBLK, H), lambda i: (i, 0))
    cblk = pl.BlockSpec((NC * NS, DBLK), lambda i: (0, i))
    wblk = pl.BlockSpec((H, H), lambda i: (0, 0))
    return pl.pallas_call(
        _dense_kernel,
        grid=grid,
        in_specs=[blk, cblk, blk, wblk, wblk,
                  pl.BlockSpec((1, H), lambda i: (0, 0))],
        out_specs=blk,
        out_shape=jax.ShapeDtypeStruct((NPAD, H), jnp.float32),
    )(acc, cnt, x, a, cm, d)


def _mlp_kernel(u_ref, i_ref, w1u_ref, w1i_ref, b1_ref, w2_ref, b2_ref, o_ref):
    h = (jnp.dot(u_ref[...], w1u_ref[...], preferred_element_type=jnp.float32)
         + jnp.dot(i_ref[...], w1i_ref[...], preferred_element_type=jnp.float32)
         + b1_ref[...])
    h = jnp.maximum(h, 0.0)
    o_ref[...] = (jnp.dot(h, w2_ref[...], preferred_element_type=jnp.float32)
                  + b2_ref[...])


def _mlp(uh, ih, w1u, w1i, b1, w2t, b2):
    grid = (BPAD // MBLK,)
    blk = pl.BlockSpec((MBLK, H), lambda i: (i, 0))
    return pl.pallas_call(
        _mlp_kernel,
        grid=grid,
        in_specs=[blk, blk,
                  pl.BlockSpec((H, H), lambda i: (0, 0)),
                  pl.BlockSpec((H, H), lambda i: (0, 0)),
                  pl.BlockSpec((1, H), lambda i: (0, 0)),
                  pl.BlockSpec((H, 8), lambda i: (0, 0)),
                  pl.BlockSpec((1, 8), lambda i: (0, 0))],
        out_specs=pl.BlockSpec((MBLK, 8), lambda i: (i, 0)),
        out_shape=jax.ShapeDtypeStruct((BPAD, 8), jnp.float32),
    )(uh, ih, w1u, w1i, b1, w2t, b2)


def _pad_table(x):
    z = jnp.zeros((PADW, H), jnp.float32)
    return jnp.concatenate([x[:HALF], z, x[HALF:], z], axis=0)


def _pad_groups(idx, ngroups, fill):
    pad = jnp.full((ngroups * G - idx.shape[0],), fill, jnp.int32)
    return jnp.concatenate([idx, pad]).reshape(ngroups, G)


def kernel(edge_index, edge_label_index, ue, ie, uWl, uWr, ub, ug, ubb,
           iWl, iWr, ib, ig, ibb, fcW1, fcb1, fcW2, fcb2):
    u2d = _pad_groups(edge_index[0].astype(jnp.int32), EGP, N)
    i2d = _pad_groups(edge_index[1].astype(jnp.int32), EGP, N)
    eli_u = _pad_groups(edge_label_index[0].astype(jnp.int32), BG, 0)
    eli_i = _pad_groups(edge_label_index[1].astype(jnp.int32), BG, 0)

    zeros = jnp.zeros((ZS, H), jnp.float32)

    upad = _pad_table(ue)
    ipad = _pad_table(ie)

    cntu, cnti = _cnt(u2d, i2d)

    gscale = 1.0 / jnp.sqrt(1.0 + EPS)
    for l in range(NLAYERS):
        accu = _agg(i2d, u2d, ipad, zeros).reshape(NPAD, H)
        acci = _agg(u2d, i2d, upad, zeros).reshape(NPAD, H)
        gu = ug[l] * gscale
        gi = ig[l] * gscale
        au = uWl[l].T * gu[None, :]
        cu = uWr[l].T * gu[None, :]
        du = (ub[l] * gu + ubb[l])[None, :]
        ai = iWl[l].T * gi[None, :]
        ci = iWr[l].T * gi[None, :]
        di = (ib[l] * gi + ibb[l])[None, :]
        new_u = _dense(accu, cntu, upad, au, cu, du)
        new_i = _dense(acci, cnti, ipad, ai, ci, di)
        upad, ipad = new_u, new_i

    uh, ih = _fgather(eli_u, eli_i, upad, ipad)

    w1u = fcW1[:, :H].T
    w1i = fcW1[:, H:].T
    b1 = fcb1[None, :]
    w2t = jnp.zeros((H, 8), jnp.float32).at[:, :4].set(fcW2.T)
    b2 = jnp.zeros((1, 8), jnp.float32).at[0, :4].set(fcb2)
    out8 = _mlp(uh, ih, w1u, w1i, b1, w2t, b2)
    return out8[:B, :4]


# speedup vs baseline: 18.4483x; 18.4483x over previous
import functools

import jax
import jax.numpy as jnp
from jax import lax
from jax.experimental import pallas as pl
from jax.experimental.pallas import tpu as pltpu
from jax.experimental.pallas import tpu_sc as plsc

N = 50000
H = 64
E = 800000
B = 200000
NLAYERS = 2
EPS = 1e-5

HALF = 25000
TRASH = HALF
HPAD = 25088
NPAD = 2 * HPAD
PADW = HPAD - HALF

NC = 2
NS = 16
G = 128
EG = E // G
GSUB = 392
EGP = GSUB * NS
SUPG = 28
NSUP = GSUB // SUPG
ZS = HPAD // NS
CW = 16

BG = 1568
BPAD = BG * G
FG_PW = BG // (NC * NS)

_mesh = plsc.VectorSubcoreMesh(core_axis_name="c", subcore_axis_name="s")
_SC_PARAMS = pltpu.CompilerParams(use_tc_tiling_on_sc=False)
_SC_PARAMS_NL = pltpu.CompilerParams(use_tc_tiling_on_sc=False,
                                     needs_layout_passes=False)


def _remap_src_row(row_ref):
    for i in range(G // 16):
        v = row_ref[pl.ds(i * 16, 16)]
        row_ref[pl.ds(i * 16, 16)] = jnp.where(v >= HALF, v + PADW, v)


def _remap_dst_row(row_ref, c):
    base = c * HALF
    for i in range(G // 16):
        v = row_ref[pl.ds(i * 16, 16)] - base
        ok = (v >= 0) & (v < HALF)
        row_ref[pl.ds(i * 16, 16)] = jnp.where(ok, v, TRASH)


def _remap_pair_row(src_ref, dst_ref, c):
    base = c * HALF
    for i in range(G // 16):
        sl = pl.ds(i * 16, 16)
        d = dst_ref[sl] - base
        ok = (d >= 0) & (d < HALF)
        dst_ref[sl] = jnp.where(ok, d, TRASH)
        v = src_ref[sl]
        v = jnp.where(v >= HALF, v + PADW, v)
        src_ref[sl] = jnp.where(ok, v, TRASH)
    return None


def _zero_and_barrier(zeros_hbm, acc_sh, s):
    pltpu.sync_copy(zeros_hbm, acc_sh.at[pl.ds(s * ZS, ZS)])
    plsc.subcore_barrier()


def _dump_half(acc_sh, out_hbm, c, s):
    pltpu.sync_copy(acc_sh.at[pl.ds(s * ZS, ZS)],
                    out_hbm.at[c, pl.ds(s * ZS, ZS)])


def _agg_body(src_hbm, dst_hbm, table_hbm, zeros_hbm, out_hbm,
              acc_sh, sidx, ldst, rows,
              isem0, isem1, gsem0, gsem1, ssem0, ssem1):
    c = lax.axis_index("c")
    s = lax.axis_index("s")
    _zero_and_barrier(zeros_hbm, acc_sh, s)
    isems = (isem0, isem1)
    gsems = (gsem0, gsem1)
    ssems = (ssem0, ssem1)
    g0row = s * GSUB

    def idx_start(p, pp):
        base = g0row + p * SUPG
        pltpu.async_copy(src_hbm.at[pl.ds(base, SUPG)], sidx.at[pp],
                         isems[pp])
        pltpu.async_copy(dst_hbm.at[pl.ds(base, SUPG)], ldst.at[pp],
                         isems[pp])

    def idx_wait(p, pp):
        base = g0row + p * SUPG
        pltpu.make_async_copy(src_hbm.at[pl.ds(base, SUPG)], sidx.at[pp],
                              isems[pp]).wait()
        pltpu.make_async_copy(dst_hbm.at[pl.ds(base, SUPG)], ldst.at[pp],
                              isems[pp]).wait()

    def gather_start(rr, pp, r):
        pltpu.async_copy(table_hbm.at[sidx.at[pp, r]], rows.at[rr],
                         gsems[rr])

    def gather_wait(rr, pp, r):
        pltpu.make_async_copy(table_hbm.at[sidx.at[pp, r]], rows.at[rr],
                              gsems[rr]).wait()

    def scatter_start(rr, pp, r):
        pltpu.async_copy(rows.at[rr], acc_sh.at[ldst.at[pp, r]], ssems[rr],
                         add=True)

    def scatter_wait(rr, pp, r):
        pltpu.make_async_copy(rows.at[rr], acc_sh.at[ldst.at[pp, r]],
                              ssems[rr]).wait()

    def do_chunk(p, pp):
        idx_wait(p, pp)

        def rbody(r, carry):
            _remap_src_row(sidx.at[pp, r])
            _remap_dst_row(ldst.at[pp, r], c)
            return carry

        lax.fori_loop(0, SUPG, rbody, 0)

        def gbody(r0, carry):
            for rr in (0, 1):
                r = r0 * 2 + rr

                @pl.when(r0 >= 1)
                def _():
                    scatter_wait(rr, pp, r - 2)

                gather_start(rr, pp, r)
                gather_wait(rr, pp, r)
                scatter_start(rr, pp, r)
            return carry

        lax.fori_loop(0, SUPG // 2, gbody, 0)
        scatter_wait(0, pp, SUPG - 2)
        scatter_wait(1, pp, SUPG - 1)

        @pl.when(p < NSUP - 2)
        def _():
            idx_start(p + 2, pp)

    idx_start(0, 0)
    idx_start(1, 1)

    def pbody(p0, carry):
        do_chunk(p0 * 2, 0)
        do_chunk(p0 * 2 + 1, 1)
        return carry

    lax.fori_loop(0, NSUP // 2, pbody, 0)
    plsc.subcore_barrier()
    _dump_half(acc_sh, out_hbm, c, s)


_agg = functools.partial(
    pl.kernel,
    out_type=jax.ShapeDtypeStruct((NC, HPAD, H), jnp.float32),
    mesh=_mesh,
    compiler_params=_SC_PARAMS,
    scratch_types=[
        pltpu.VMEM_SHARED((HPAD, H), jnp.float32),
        pltpu.VMEM((2, SUPG, G), jnp.int32),
        pltpu.VMEM((2, SUPG, G), jnp.int32),
        pltpu.VMEM((2, G, H), jnp.float32),
    ] + [pltpu.SemaphoreType.DMA] * 6,
)(_agg_body)


def _cnt_body(udst_hbm, idst_hbm, outu_hbm, outi_hbm, hist, didx):
    c = lax.axis_index("c")
    s = lax.axis_index("s")
    w = s * NC + c
    ones16 = jnp.ones((16,), jnp.float32)
    gpw = EGP // (NC * NS)

    for dst_hbm, out_hbm in ((udst_hbm, outu_hbm), (idst_hbm, outi_hbm)):
        def zbody(i, carry):
            hist[pl.ds(i * 16, 16)] = jnp.zeros((16,), jnp.float32)
            return carry

        lax.fori_loop(0, NPAD // 16, zbody, 0)

        def cbody(p, carry):
            pltpu.sync_copy(dst_hbm.at[pl.ds(w * gpw + p * SUPG, SUPG)],
                            didx)

            def rbody(r, carry2):
                for i in range(G // 16):
                    v = didx[r, pl.ds(i * 16, 16)]
                    v = jnp.where(v >= HALF, v + PADW, v)
                    plsc.addupdate_scatter(hist, [v], ones16)
                return carry2

            lax.fori_loop(0, SUPG, rbody, 0)
            return carry

        lax.fori_loop(0, gpw // SUPG, cbody, 0)
        pltpu.sync_copy(hist, out_hbm.at[w])


_cnt = functools.partial(
    pl.kernel,
    out_type=(jax.ShapeDtypeStruct((NC * NS, NPAD), jnp.float32),
              jax.ShapeDtypeStruct((NC * NS, NPAD), jnp.float32)),
    mesh=_mesh,
    compiler_params=_SC_PARAMS_NL,
    scratch_types=[
        pltpu.VMEM((NPAD,), jnp.float32),
        pltpu.VMEM((SUPG, G), jnp.int32),
    ],
)(_cnt_body)


def _fgather_body(ui_hbm, ii_hbm, utab_hbm, itab_hbm, uh_hbm, ih_hbm,
                  eidx, rows,
                  gsem0, gsem1, gsem2, gsem3, osem0, osem1, osem2, osem3):
    c = lax.axis_index("c")
    s = lax.axis_index("s")
    w = s * NC + c
    gsems = (gsem0, gsem1, gsem2, gsem3)
    osems = (osem0, osem1, osem2, osem3)
    tabs = (utab_hbm, itab_hbm)
    outs = (uh_hbm, ih_hbm)
    g0row = w * FG_PW

    pltpu.sync_copy(ui_hbm.at[pl.ds(g0row, FG_PW)], eidx.at[0])
    pltpu.sync_copy(ii_hbm.at[pl.ds(g0row, FG_PW)], eidx.at[1])

    def rbody(r, carry):
        _remap_src_row(eidx.at[0, r])
        _remap_src_row(eidx.at[1, r])
        return carry

    lax.fori_loop(0, FG_PW, rbody, 0)

    def gather_start(d, bi, k):
        pltpu.async_copy(tabs[d].at[eidx.at[d, k]], rows.at[bi], gsems[bi])

    def gather_wait(d, bi, k):
        pltpu.make_async_copy(tabs[d].at[eidx.at[d, k]], rows.at[bi],
                              gsems[bi]).wait()

    def write_start(d, bi, k):
        pltpu.async_copy(rows.at[bi], outs[d].at[pl.ds((g0row + k) * G, G)],
                         osems[bi])

    def write_wait(d, bi, k):
        pltpu.make_async_copy(rows.at[bi],
                              outs[d].at[pl.ds((g0row + k) * G, G)],
                              osems[bi]).wait()

    def body(k0, carry):
        for kk in (0, 1):
            k = k0 * 2 + kk
            for d in (0, 1):
                bi = 2 * d + kk
                ob = 2 * d + (1 - kk)

                @pl.when(k0 >= 1)
                def _():
                    write_wait(d, bi, k - 2)

                gather_start(d, bi, k)

                if kk == 0:
                    @pl.when(k0 >= 1)
                    def _():
                        gather_wait(d, ob, k - 1)
                        write_start(d, ob, k - 1)
                else:
                    gather_wait(d, ob, k - 1)
                    write_start(d, ob, k - 1)
        return carry

    lax.fori_loop(0, 24, body, 0)
    for d in (0, 1):
        gather_wait(d, 2 * d + 1, 47)
        write_start(d, 2 * d + 1, 47)
        write_wait(d, 2 * d, 46)
        gather_start(d, 2 * d, 48)
    for d in (0, 1):
        gather_wait(d, 2 * d, 48)
        write_start(d, 2 * d, 48)
    for d in (0, 1):
        write_wait(d, 2 * d + 1, 47)
        write_wait(d, 2 * d, 48)


_fgather = functools.partial(
    pl.kernel,
    out_type=(jax.ShapeDtypeStruct((BPAD, H), jnp.float32),
              jax.ShapeDtypeStruct((BPAD, H), jnp.float32)),
    mesh=_mesh,
    compiler_params=_SC_PARAMS,
    scratch_types=[
        pltpu.VMEM((2, FG_PW, G), jnp.int32),
        pltpu.VMEM((4, G, H), jnp.float32),
    ] + [pltpu.SemaphoreType.DMA] * 8,
)(_fgather_body)


DBLK = 512
MBLK = 1024


def _dense_kernel(acc_ref, cnt_ref, x_ref, a_ref, c_ref, d_ref, o_ref):
    cnt = jnp.sum(cnt_ref[...], axis=0)[:, None]
    mean = acc_ref[...] / jnp.maximum(cnt, 1.0)
    y = (jnp.dot(mean, a_ref[...], preferred_element_type=jnp.float32)
         + jnp.dot(x_ref[...], c_ref[...], preferred_element_type=jnp.float32)
         + d_ref[...])
    o_ref[...] = jnp.maximum(y, 0.0)


def _dense(acc, cnt, x, a, cm, d):
    grid = (NPAD // DBLK,)
    blk = pl.BlockSpec((DBLK, H), lambda i: (i, 0))
    cblk = pl.BlockSpec((NC * NS, DBLK), lambda i: (0, i))
    wblk = pl.BlockSpec((H, H), lambda i: (0, 0))
    return pl.pallas_call(
        _dense_kernel,
        grid=grid,
        in_specs=[blk, cblk, blk, wblk, wblk,
                  pl.BlockSpec((1, H), lambda i: (0, 0))],
        out_specs=blk,
        out_shape=jax.ShapeDtypeStruct((NPAD, H), jnp.float32),
    )(acc, cnt, x, a, cm, d)


def _mlp_kernel(u_ref, i_ref, w1u_ref, w1i_ref, b1_ref, w2_ref, b2_ref, o_ref):
    h = (jnp.dot(u_ref[...], w1u_ref[...], preferred_element_type=jnp.float32)
         + jnp.dot(i_ref[...], w1i_ref[...], preferred_element_type=jnp.float32)
         + b1_ref[...])
    h = jnp.maximum(h, 0.0)
    o_ref[...] = (jnp.dot(h, w2_ref[...], preferred_element_type=jnp.float32)
                  + b2_ref[...])


def _mlp(uh, ih, w1u, w1i, b1, w2t, b2):
    grid = (BPAD // MBLK,)
    blk = pl.BlockSpec((MBLK, H), lambda i: (i, 0))
    return pl.pallas_call(
        _mlp_kernel,
        grid=grid,
        in_specs=[blk, blk,
                  pl.BlockSpec((H, H), lambda i: (0, 0)),
                  pl.BlockSpec((H, H), lambda i: (0, 0)),
                  pl.BlockSpec((1, H), lambda i: (0, 0)),
                  pl.BlockSpec((H, 8), lambda i: (0, 0)),
                  pl.BlockSpec((1, 8), lambda i: (0, 0))],
        out_specs=pl.BlockSpec((MBLK, 8), lambda i: (i, 0)),
        out_shape=jax.ShapeDtypeStruct((BPAD, 8), jnp.float32),
    )(uh, ih, w1u, w1i, b1, w2t, b2)


def _pad_table(x):
    z = jnp.zeros((PADW, H), jnp.float32)
    return jnp.concatenate([x[:HALF], z, x[HALF:], z], axis=0)


def _pad_groups(idx, ngroups, fill):
    pad = jnp.full((ngroups * G - idx.shape[0],), fill, jnp.int32)
    return jnp.concatenate([idx, pad]).reshape(ngroups, G)


def kernel(edge_index, edge_label_index, ue, ie, uWl, uWr, ub, ug, ubb,
           iWl, iWr, ib, ig, ibb, fcW1, fcb1, fcW2, fcb2):
    u2d = _pad_groups(edge_index[0].astype(jnp.int32), EGP, N)
    i2d = _pad_groups(edge_index[1].astype(jnp.int32), EGP, N)
    eli_u = _pad_groups(edge_label_index[0].astype(jnp.int32), BG, 0)
    eli_i = _pad_groups(edge_label_index[1].astype(jnp.int32), BG, 0)

    zeros = jnp.zeros((ZS, H), jnp.float32)

    upad = _pad_table(ue)
    ipad = _pad_table(ie)

    cntu, cnti = _cnt(u2d, i2d)

    gscale = 1.0 / jnp.sqrt(1.0 + EPS)
    for l in range(NLAYERS):
        accu = _agg(i2d, u2d, ipad, zeros).reshape(NPAD, H)
        acci = _agg(u2d, i2d, upad, zeros).reshape(NPAD, H)
        gu = ug[l] * gscale
        gi = ig[l] * gscale
        au = uWl[l].T * gu[None, :]
        cu = uWr[l].T * gu[None, :]
        du = (ub[l] * gu + ubb[l])[None, :]
        ai = iWl[l].T * gi[None, :]
        ci = iWr[l].T * gi[None, :]
        di = (ib[l] * gi + ibb[l])[None, :]
        new_u = _dense(accu, cntu, upad, au, cu, du)
        new_i = _dense(acci, cnti, ipad, ai, ci, di)
        upad, ipad = new_u, new_i

    uh, ih = _fgather(eli_u, eli_i, upad, ipad)

    w1u = fcW1[:, :H].T
    w1i = fcW1[:, H:].T
    b1 = fcb1[None, :]
    w2t = jnp.zeros((H, 8), jnp.float32).at[:, :4].set(fcW2.T)
    b2 = jnp.zeros((1, 8), jnp.float32).at[0, :4].set(fcb2)
    out8 = _mlp(uh, ih, w1u, w1i, b1, w2t, b2)
    return out8[:B, :4]


# speedup vs baseline: 18.4518x; 1.0002x over previous
import functools

import jax
import jax.numpy as jnp
from jax import lax
from jax.experimental import pallas as pl
from jax.experimental.pallas import tpu as pltpu
from jax.experimental.pallas import tpu_sc as plsc

N = 50000
H = 64
E = 800000
B = 200000
NLAYERS = 2
EPS = 1e-5

HALF = 25000
TRASH = HALF
HPAD = 25088
NPAD = 2 * HPAD
PADW = HPAD - HALF

NC = 2
NS = 16
G = 128
EG = E // G
GSUB = 392
EGP = GSUB * NS
SUPG = 28
NSUP = GSUB // SUPG
ZS = HPAD // NS

BG = 1568
BPAD = BG * G
FG_PW = BG // (NC * NS)

_mesh = plsc.VectorSubcoreMesh(core_axis_name="c", subcore_axis_name="s")
_SC_PARAMS = pltpu.CompilerParams(use_tc_tiling_on_sc=False)
_SC_PARAMS_NL = pltpu.CompilerParams(use_tc_tiling_on_sc=False,
                                     needs_layout_passes=False)


def _remap_src_row(row_ref):
    for i in range(G // 16):
        v = row_ref[pl.ds(i * 16, 16)]
        row_ref[pl.ds(i * 16, 16)] = jnp.where(v >= HALF, v + PADW, v)


def _remap_dst_row(row_ref, c):
    base = c * HALF
    for i in range(G // 16):
        v = row_ref[pl.ds(i * 16, 16)] - base
        ok = (v >= 0) & (v < HALF)
        row_ref[pl.ds(i * 16, 16)] = jnp.where(ok, v, TRASH)


def _zero_and_barrier(zeros_hbm, acc_sh, s):
    pltpu.sync_copy(zeros_hbm, acc_sh.at[pl.ds(s * ZS, ZS)])
    plsc.subcore_barrier()


def _dump_half(acc_sh, out_hbm, c, s):
    pltpu.sync_copy(acc_sh.at[pl.ds(s * ZS, ZS)],
                    out_hbm.at[c, pl.ds(s * ZS, ZS)])


def _agg_body(src_hbm, dst_hbm, table_hbm, zeros_hbm, out_hbm,
              acc_sh, sidx, ldst, rows,
              isem0, isem1, gsem0, gsem1, ssem0, ssem1):
    c = lax.axis_index("c")
    s = lax.axis_index("s")
    _zero_and_barrier(zeros_hbm, acc_sh, s)
    isems = (isem0, isem1)
    gsems = (gsem0, gsem1)
    ssems = (ssem0, ssem1)
    g0row = s * GSUB

    def idx_start(p, pp):
        base = g0row + p * SUPG
        pltpu.async_copy(src_hbm.at[pl.ds(base, SUPG)], sidx.at[pp],
                         isems[pp])
        pltpu.async_copy(dst_hbm.at[pl.ds(base, SUPG)], ldst.at[pp],
                         isems[pp])

    def idx_wait(p, pp):
        base = g0row + p * SUPG
        pltpu.make_async_copy(src_hbm.at[pl.ds(base, SUPG)], sidx.at[pp],
                              isems[pp]).wait()
        pltpu.make_async_copy(dst_hbm.at[pl.ds(base, SUPG)], ldst.at[pp],
                              isems[pp]).wait()

    def gather_start(rr, pp, r):
        pltpu.async_copy(table_hbm.at[sidx.at[pp, r]], rows.at[rr],
                         gsems[rr])

    def gather_wait(rr, pp, r):
        pltpu.make_async_copy(table_hbm.at[sidx.at[pp, r]], rows.at[rr],
                              gsems[rr]).wait()

    def scatter_start(rr, pp, r):
        pltpu.async_copy(rows.at[rr], acc_sh.at[ldst.at[pp, r]], ssems[rr],
                         add=True)

    def scatter_wait(rr, pp, r):
        pltpu.make_async_copy(rows.at[rr], acc_sh.at[ldst.at[pp, r]],
                              ssems[rr]).wait()

    def do_chunk(p, pp):
        idx_wait(p, pp)

        def rbody(r, carry):
            _remap_src_row(sidx.at[pp, r])
            _remap_dst_row(ldst.at[pp, r], c)
            return carry

        lax.fori_loop(0, SUPG, rbody, 0)

        def gbody(r0, carry):
            for rr in (0, 1):
                r = r0 * 2 + rr

                @pl.when(r0 >= 1)
                def _():
                    scatter_wait(rr, pp, r - 2)

                gather_start(rr, pp, r)
                gather_wait(rr, pp, r)
                scatter_start(rr, pp, r)
            return carry

        lax.fori_loop(0, SUPG // 2, gbody, 0)
        scatter_wait(0, pp, SUPG - 2)
        scatter_wait(1, pp, SUPG - 1)

        @pl.when(p < NSUP - 2)
        def _():
            idx_start(p + 2, pp)

    idx_start(0, 0)
    idx_start(1, 1)

    def pbody(p0, carry):
        do_chunk(p0 * 2, 0)
        do_chunk(p0 * 2 + 1, 1)
        return carry

    lax.fori_loop(0, NSUP // 2, pbody, 0)
    plsc.subcore_barrier()
    _dump_half(acc_sh, out_hbm, c, s)


_agg = functools.partial(
    pl.kernel,
    out_type=jax.ShapeDtypeStruct((NC, HPAD, H), jnp.float32),
    mesh=_mesh,
    compiler_params=_SC_PARAMS,
    scratch_types=[
        pltpu.VMEM_SHARED((HPAD, H), jnp.float32),
        pltpu.VMEM((2, SUPG, G), jnp.int32),
        pltpu.VMEM((2, SUPG, G), jnp.int32),
        pltpu.VMEM((2, G, H), jnp.float32),
    ] + [pltpu.SemaphoreType.DMA] * 6,
)(_agg_body)


def _cnt_body(udst_hbm, idst_hbm, outu_hbm, outi_hbm, hist, didx):
    c = lax.axis_index("c")
    s = lax.axis_index("s")
    w = s * NC + c
    ones16 = jnp.ones((16,), jnp.float32)
    gpw = EGP // (NC * NS)

    for dst_hbm, out_hbm in ((udst_hbm, outu_hbm), (idst_hbm, outi_hbm)):
        def zbody(i, carry):
            hist[pl.ds(i * 16, 16)] = jnp.zeros((16,), jnp.float32)
            return carry

        lax.fori_loop(0, NPAD // 16, zbody, 0)

        def cbody(p, carry):
            pltpu.sync_copy(dst_hbm.at[pl.ds(w * gpw + p * SUPG, SUPG)],
                            didx)

            def rbody(r, carry2):
                for i in range(G // 16):
                    v = didx[r, pl.ds(i * 16, 16)]
                    v = jnp.where(v >= HALF, v + PADW, v)
                    plsc.addupdate_scatter(hist, [v], ones16)
                return carry2

            lax.fori_loop(0, SUPG, rbody, 0)
            return carry

        lax.fori_loop(0, gpw // SUPG, cbody, 0)
        pltpu.sync_copy(hist, out_hbm.at[w])


_cnt = functools.partial(
    pl.kernel,
    out_type=(jax.ShapeDtypeStruct((NC * NS, NPAD), jnp.float32),
              jax.ShapeDtypeStruct((NC * NS, NPAD), jnp.float32)),
    mesh=_mesh,
    compiler_params=_SC_PARAMS_NL,
    scratch_types=[
        pltpu.VMEM((NPAD,), jnp.float32),
        pltpu.VMEM((SUPG, G), jnp.int32),
    ],
)(_cnt_body)


def _fgather_body(ui_hbm, ii_hbm, utab_hbm, itab_hbm, uh_hbm, ih_hbm,
                  eidx, rows,
                  gsem0, gsem1, gsem2, gsem3, osem0, osem1, osem2, osem3):
    c = lax.axis_index("c")
    s = lax.axis_index("s")
    w = s * NC + c
    gsems = (gsem0, gsem1, gsem2, gsem3)
    osems = (osem0, osem1, osem2, osem3)
    tabs = (utab_hbm, itab_hbm)
    outs = (uh_hbm, ih_hbm)
    g0row = w * FG_PW

    pltpu.sync_copy(ui_hbm.at[pl.ds(g0row, FG_PW)], eidx.at[0])
    pltpu.sync_copy(ii_hbm.at[pl.ds(g0row, FG_PW)], eidx.at[1])

    def rbody(r, carry):
        _remap_src_row(eidx.at[0, r])
        _remap_src_row(eidx.at[1, r])
        return carry

    lax.fori_loop(0, FG_PW, rbody, 0)

    def gather_start(d, bi, k):
        pltpu.async_copy(tabs[d].at[eidx.at[d, k]], rows.at[bi], gsems[bi])

    def gather_wait(d, bi, k):
        pltpu.make_async_copy(tabs[d].at[eidx.at[d, k]], rows.at[bi],
                              gsems[bi]).wait()

    def write_start(d, bi, k):
        pltpu.async_copy(rows.at[bi], outs[d].at[pl.ds((g0row + k) * G, G)],
                         osems[bi])

    def write_wait(d, bi, k):
        pltpu.make_async_copy(rows.at[bi],
                              outs[d].at[pl.ds((g0row + k) * G, G)],
                              osems[bi]).wait()

    def body(k0, carry):
        for kk in (0, 1):
            k = k0 * 2 + kk
            for d in (0, 1):
                bi = 2 * d + kk
                ob = 2 * d + (1 - kk)

                @pl.when(k0 >= 1)
                def _():
                    write_wait(d, bi, k - 2)

                gather_start(d, bi, k)

                if kk == 0:
                    @pl.when(k0 >= 1)
                    def _():
                        gather_wait(d, ob, k - 1)
                        write_start(d, ob, k - 1)
                else:
                    gather_wait(d, ob, k - 1)
                    write_start(d, ob, k - 1)
        return carry

    lax.fori_loop(0, 24, body, 0)
    for d in (0, 1):
        gather_wait(d, 2 * d + 1, 47)
        write_start(d, 2 * d + 1, 47)
        write_wait(d, 2 * d, 46)
        gather_start(d, 2 * d, 48)
    for d in (0, 1):
        gather_wait(d, 2 * d, 48)
        write_start(d, 2 * d, 48)
    for d in (0, 1):
        write_wait(d, 2 * d + 1, 47)
        write_wait(d, 2 * d, 48)


_fgather = functools.partial(
    pl.kernel,
    out_type=(jax.ShapeDtypeStruct((BPAD, H), jnp.float32),
              jax.ShapeDtypeStruct((BPAD, H), jnp.float32)),
    mesh=_mesh,
    compiler_params=_SC_PARAMS,
    scratch_types=[
        pltpu.VMEM((2, FG_PW, G), jnp.int32),
        pltpu.VMEM((4, G, H), jnp.float32),
    ] + [pltpu.SemaphoreType.DMA] * 8,
)(_fgather_body)


DBLK = 512
MBLK = 1024


def _dense_kernel(acc_ref, cnt_ref, x_ref, a_ref, c_ref, d_ref, o_ref):
    cnt = jnp.sum(cnt_ref[...], axis=0)[:, None]
    mean = acc_ref[...] / jnp.maximum(cnt, 1.0)
    y = (jnp.dot(mean, a_ref[...], preferred_element_type=jnp.float32)
         + jnp.dot(x_ref[...], c_ref[...], preferred_element_type=jnp.float32)
         + d_ref[...])
    o_ref[...] = jnp.maximum(y, 0.0)


def _dense(acc, cnt, x, a, cm, d):
    grid = (NPAD // DBLK,)
    blk = pl.BlockSpec((DBLK, H), lambda i: (i, 0))
    cblk = pl.BlockSpec((NC * NS, DBLK), lambda i: (0, i))
    wblk = pl.BlockSpec((H, H), lambda i: (0, 0))
    return pl.pallas_call(
        _dense_kernel,
        grid=grid,
        in_specs=[blk, cblk, blk, wblk, wblk,
                  pl.BlockSpec((1, H), lambda i: (0, 0))],
        out_specs=blk,
        out_shape=jax.ShapeDtypeStruct((NPAD, H), jnp.float32),
    )(acc, cnt, x, a, cm, d)


def _mlp_kernel(u_ref, i_ref, w1u_ref, w1i_ref, b1_ref, w2_ref, b2_ref, o_ref):
    h = (jnp.dot(u_ref[...], w1u_ref[...], preferred_element_type=jnp.float32)
         + jnp.dot(i_ref[...], w1i_ref[...], preferred_element_type=jnp.float32)
         + b1_ref[...])
    h = jnp.maximum(h, 0.0)
    o_ref[...] = (jnp.dot(h, w2_ref[...], preferred_element_type=jnp.float32)
                  + b2_ref[...])


def _mlp(uh, ih, w1u, w1i, b1, w2t, b2):
    grid = (BPAD // MBLK,)
    blk = pl.BlockSpec((MBLK, H), lambda i: (i, 0))
    return pl.pallas_call(
        _mlp_kernel,
        grid=grid,
        in_specs=[blk, blk,
                  pl.BlockSpec((H, H), lambda i: (0, 0)),
                  pl.BlockSpec((H, H), lambda i: (0, 0)),
                  pl.BlockSpec((1, H), lambda i: (0, 0)),
                  pl.BlockSpec((H, 8), lambda i: (0, 0)),
                  pl.BlockSpec((1, 8), lambda i: (0, 0))],
        out_specs=pl.BlockSpec((MBLK, 8), lambda i: (i, 0)),
        out_shape=jax.ShapeDtypeStruct((BPAD, 8), jnp.float32),
    )(uh, ih, w1u, w1i, b1, w2t, b2)


def _pad_table(x):
    z = jnp.zeros((PADW, H), jnp.float32)
    return jnp.concatenate([x[:HALF], z, x[HALF:], z], axis=0)


def _pad_groups(idx, ngroups, fill):
    pad = jnp.full((ngroups * G - idx.shape[0],), fill, jnp.int32)
    return jnp.concatenate([idx, pad]).reshape(ngroups, G)


def kernel(edge_index, edge_label_index, ue, ie, uWl, uWr, ub, ug, ubb,
           iWl, iWr, ib, ig, ibb, fcW1, fcb1, fcW2, fcb2):
    u2d = _pad_groups(edge_index[0].astype(jnp.int32), EGP, N)
    i2d = _pad_groups(edge_index[1].astype(jnp.int32), EGP, N)
    eli_u = _pad_groups(edge_label_index[0].astype(jnp.int32), BG, 0)
    eli_i = _pad_groups(edge_label_index[1].astype(jnp.int32), BG, 0)

    zeros = jnp.zeros((ZS, H), jnp.float32)

    upad = _pad_table(ue)
    ipad = _pad_table(ie)

    cntu, cnti = _cnt(u2d, i2d)

    gscale = 1.0 / jnp.sqrt(1.0 + EPS)
    for l in range(NLAYERS):
        accu = _agg(i2d, u2d, ipad, zeros).reshape(NPAD, H)
        acci = _agg(u2d, i2d, upad, zeros).reshape(NPAD, H)
        gu = ug[l] * gscale
        gi = ig[l] * gscale
        au = uWl[l].T * gu[None, :]
        cu = uWr[l].T * gu[None, :]
        du = (ub[l] * gu + ubb[l])[None, :]
        ai = iWl[l].T * gi[None, :]
        ci = iWr[l].T * gi[None, :]
        di = (ib[l] * gi + ibb[l])[None, :]
        new_u = _dense(accu, cntu, upad, au, cu, du)
        new_i = _dense(acci, cnti, ipad, ai, ci, di)
        upad, ipad = new_u, new_i

    uh, ih = _fgather(eli_u, eli_i, upad, ipad)

    w1u = fcW1[:, :H].T
    w1i = fcW1[:, H:].T
    b1 = fcb1[None, :]
    w2t = jnp.zeros((H, 8), jnp.float32).at[:, :4].set(fcW2.T)
    b2 = jnp.zeros((1, 8), jnp.float32).at[0, :4].set(fcb2)
    out8 = _mlp(uh, ih, w1u, w1i, b1, w2t, b2)
    return out8[:B, :4]
